# baseline (speedup 1.0000x reference)
import functools
import jax
import jax.numpy as jnp
from jax import lax
from jax.experimental import pallas as pl
from jax.experimental.pallas import tpu as pltpu
from jax.experimental.pallas import tpu_sc as plsc

D_MODEL = 64
NUM_WORKERS = 32
BLK = 128


def _make_gather(seq: int, batch: int, vocab_pairs: int):
    n_bb = batch // BLK
    assert n_bb == NUM_WORKERS
    mesh = plsc.VectorSubcoreMesh(core_axis_name="c", subcore_axis_name="s")

    @functools.partial(
        pl.kernel,
        mesh=mesh,
        out_type=jax.ShapeDtypeStruct((seq, D_MODEL, batch), jnp.float32),
        scratch_types=[
            pltpu.VMEM((BLK,), jnp.int32),
            pltpu.VMEM((BLK,), jnp.int32),
            pltpu.VMEM((BLK,), jnp.int32),
            pltpu.VMEM((BLK, 2 * D_MODEL), jnp.float32),
            pltpu.VMEM((D_MODEL, BLK), jnp.float32),
            pltpu.SemaphoreType.DMA,
            pltpu.SemaphoreType.DMA,
        ],
        compiler_params=pltpu.CompilerParams(needs_layout_passes=False),
    )
    def gather_kernel(idx_hbm, table_hbm, out_hbm,
                      idx_v, pair_idx_v, half_v, rows_v, tile_v,
                      sem_g, sem_w):
        wid = lax.axis_index("s") * 2 + lax.axis_index("c")
        col0 = wid * BLK

        def step(s, carry):
            pltpu.sync_copy(idx_hbm.at[s, pl.ds(col0, BLK)], idx_v)
            for q in range(BLK // 16):
                v = idx_v[pl.ds(q * 16, 16)]
                pair_idx_v[pl.ds(q * 16, 16)] = v >> 1
                half_v[pl.ds(q * 16, 16)] = (v & 1) << 6
            pltpu.async_copy(
                table_hbm.at[pair_idx_v], rows_v, sem_g
            ).wait()
            for q in range(BLK // 16):
                rows16 = lax.iota(jnp.int32, 16) + (q * 16)
                hv = half_v[pl.ds(q * 16, 16)]
                for d in range(D_MODEL):
                    tile_v[d, pl.ds(q * 16, 16)] = plsc.load_gather(
                        rows_v, [rows16, hv + d]
                    )
            pltpu.sync_copy(tile_v, out_hbm.at[s, :, pl.ds(col0, BLK)])
            return carry

        lax.fori_loop(0, seq, step, 0)

    return gather_kernel


def kernel(input, table):
    b, s = input.shape
    idx_t = input.T
    table_pairs = jnp.reshape(table, (table.shape[0] // 2, 2 * D_MODEL))
    out5 = _make_gather(s, b, table.shape[0] // 2)(idx_t, table_pairs)
    return jnp.transpose(out5, (2, 0, 1))


# speedup vs baseline: 1.3986x; 1.3986x over previous
import functools
import jax
import jax.numpy as jnp
from jax import lax
from jax.experimental import pallas as pl
from jax.experimental.pallas import tpu as pltpu
from jax.experimental.pallas import tpu_sc as plsc

D_MODEL = 64
NUM_WORKERS = 32
BLK = 128


def _make_gather(seq: int, batch: int):
    assert batch // BLK == NUM_WORKERS
    mesh = plsc.VectorSubcoreMesh(core_axis_name="c", subcore_axis_name="s")

    @functools.partial(
        pl.kernel,
        mesh=mesh,
        out_type=jax.ShapeDtypeStruct((seq, D_MODEL, batch), jnp.float32),
        scratch_types=[
            pltpu.VMEM((BLK,), jnp.int32),
            pltpu.VMEM((BLK,), jnp.int32),
            pltpu.VMEM((2, BLK, 2 * D_MODEL), jnp.float32),
            pltpu.VMEM((2, D_MODEL, BLK), jnp.float32),
            pltpu.SemaphoreType.DMA((2,)),
            pltpu.SemaphoreType.DMA((2,)),
        ],
        compiler_params=pltpu.CompilerParams(needs_layout_passes=False),
    )
    def gather_kernel(idx_hbm, table_hbm, out_hbm,
                      idx_a, idx_b, rows_v, tile_v, sem_g, sem_w):
        wid = lax.axis_index("s") * 2 + lax.axis_index("c")
        col0 = wid * BLK

        def fire(s, b, idx_v):
            pltpu.sync_copy(idx_hbm.at[s, pl.ds(col0, BLK)], idx_v)
            pltpu.async_copy(table_hbm.at[idx_v], rows_v.at[b], sem_g.at[b])

        def wait_g(b, idx_v):
            pltpu.make_async_copy(
                table_hbm.at[idx_v], rows_v.at[b], sem_g.at[b]
            ).wait()

        def extract(b):
            rows16 = [lax.iota(jnp.int32, 16) + (q * 16) for q in range(8)]

            @plsc.parallel_loop(0, D_MODEL, unroll=4)
            def body(d):
                dv = jnp.zeros((16,), jnp.int32) + d
                for q in range(8):
                    tile_v[b, d, pl.ds(q * 16, 16)] = plsc.load_gather(
                        rows_v.at[b], [rows16[q], dv]
                    )

        def fire_w(s, b):
            pltpu.async_copy(
                tile_v.at[b], out_hbm.at[s, :, pl.ds(col0, BLK)], sem_w.at[b]
            )

        def wait_w(b):
            pltpu.make_async_copy(
                tile_v.at[b], out_hbm.at[0, :, pl.ds(col0, BLK)], sem_w.at[b]
            ).wait()

        fire(0, 0, idx_a)

        def step(j, carry):
            s0 = 2 * j
            wait_g(0, idx_a)
            fire(s0 + 1, 1, idx_b)
            extract(0)
            pl.when(j > 0)(lambda: wait_w(0))
            fire_w(s0, 0)
            wait_g(1, idx_b)
            fire(s0 + 2, 0, idx_a)
            extract(1)
            pl.when(j > 0)(lambda: wait_w(1))
            fire_w(s0 + 1, 1)
            return carry

        n_main = (seq - 2) // 2
        lax.fori_loop(0, n_main, step, 0)

        s0 = 2 * n_main
        wait_g(0, idx_a)
        fire(s0 + 1, 1, idx_b)
        extract(0)
        wait_w(0)
        fire_w(s0, 0)
        wait_g(1, idx_b)
        extract(1)
        wait_w(1)
        fire_w(s0 + 1, 1)
        wait_w(0)
        wait_w(1)

    return gather_kernel


def kernel(input, table):
    b, s = input.shape
    idx_t = input.T
    table_p = jnp.pad(table, ((0, 0), (0, 2 * D_MODEL - table.shape[1])))
    out5 = _make_gather(s, b)(idx_t, table_p)
    return jnp.transpose(out5, (2, 0, 1))
